# baseline (speedup 1.0000x reference)
import functools

import jax
import jax.numpy as jnp
from jax import lax
from jax.experimental import pallas as pl
from jax.experimental.pallas import tpu as pltpu
from jax.experimental.pallas import tpu_sc as plsc

B, S, D = 4, 2048, 1024
NC, NS, L = 2, 16, 16
NW = NC * NS
S_PER_W = S // NW
R_SUB = 8
NG = S_PER_W // R_SUB
VPR = D // L
NRING = 3


def _sc_body(x_hbm, t_hbm, o_hbm, *refs):
    xbufs = [[refs[r * B + b] for b in range(B)] for r in range(NRING)]
    tbufs = list(refs[NRING * B:NRING * B + NRING])
    sems = refs[NRING * B + NRING:]
    sin = [[sems[r * B + b] for b in range(B)] for r in range(NRING)]
    sout = [[sems[NRING * B + r * B + b] for b in range(B)]
            for r in range(NRING)]
    stab = list(sems[2 * NRING * B:2 * NRING * B + NRING])

    wid = lax.axis_index("s") * NC + lax.axis_index("c")
    row0 = wid * S_PER_W

    in_d = [[None] * B for _ in range(NG)]
    out_d = [[None] * B for _ in range(NG)]
    t_d = [None] * NG

    def issue_group(g):
        r = row0 + g * R_SUB
        slot = g % NRING
        t_d[g] = pltpu.async_copy(
            t_hbm.at[pl.ds(r, R_SUB)], tbufs[slot], stab[slot])
        for b in range(B):
            in_d[g][b] = pltpu.async_copy(
                x_hbm.at[b, pl.ds(r, R_SUB)], xbufs[slot][b], sin[slot][b])

    issue_group(0)
    issue_group(1)

    for g in range(NG):
        slot = g % NRING
        t_d[g].wait()
        for b in range(B):
            in_d[g][b].wait()
        tb = tbufs[slot]
        xa, xbb, xc, xd = xbufs[slot]

        @plsc.parallel_loop(0, R_SUB * VPR, step=1, unroll=8)
        def add_loop(i):
            row = i >> 6
            col = (i & (VPR - 1)) * L
            v = tb[row, pl.ds(col, L)]
            plsc.addupdate(xa.at[row, pl.ds(col, L)], v)
            plsc.addupdate(xbb.at[row, pl.ds(col, L)], v)
            plsc.addupdate(xc.at[row, pl.ds(col, L)], v)
            plsc.addupdate(xd.at[row, pl.ds(col, L)], v)

        gn = g + 2
        if gn < NG:
            if gn - NRING >= 0:
                for b in range(B):
                    out_d[gn - NRING][b].wait()
            issue_group(gn)

        r = row0 + g * R_SUB
        for b in range(B):
            out_d[g][b] = pltpu.async_copy(
                xbufs[slot][b], o_hbm.at[b, pl.ds(r, R_SUB)], sout[slot][b])

    for g in range(max(0, NG - NRING), NG):
        for b in range(B):
            if out_d[g][b] is not None:
                out_d[g][b].wait()


_sc_call = functools.partial(
    pl.kernel,
    mesh=plsc.VectorSubcoreMesh(core_axis_name="c", subcore_axis_name="s"),
    out_type=jax.ShapeDtypeStruct((B, S, D), jnp.float32),
    scratch_types=(
        [pltpu.VMEM((R_SUB, D), jnp.float32)] * (NRING * B + NRING)
        + [pltpu.SemaphoreType.DMA] * (2 * NRING * B + NRING)
    ),
)


def kernel(x, pos_table):
    batch, seq_len, d_model = x.shape
    return _sc_call(_sc_body)(x, pos_table[:seq_len])


# speedup vs baseline: 1.0190x; 1.0190x over previous
import functools

import jax
import jax.numpy as jnp
from jax import lax
from jax.experimental import pallas as pl
from jax.experimental.pallas import tpu as pltpu
from jax.experimental.pallas import tpu_sc as plsc

B, S, D = 4, 2048, 1024
NC, NS, L = 2, 16, 16
NW = NC * NS
S_PER_W = S // NW
R_SUB = 8
NG = S_PER_W // R_SUB
VPR = D // L
NRING = 3


def _sc_body(x_hbm, t_hbm, o_hbm, *refs):
    xbufs = [[refs[r * B + b] for b in range(B)] for r in range(NRING)]
    tbufs = list(refs[NRING * B:NRING * B + NRING])
    sems = refs[NRING * B + NRING:]
    sin = [[sems[r * B + b] for b in range(B)] for r in range(NRING)]
    sout = [[sems[NRING * B + r * B + b] for b in range(B)]
            for r in range(NRING)]
    stab = list(sems[2 * NRING * B:2 * NRING * B + NRING])

    wid = lax.axis_index("s") * NC + lax.axis_index("c")
    row0 = wid * S_PER_W

    in_d = [[None] * B for _ in range(NG)]
    out_d = [[None] * B for _ in range(NG)]
    t_d = [None] * NG

    def issue_group(g):
        r = row0 + g * R_SUB
        slot = g % NRING
        t_d[g] = pltpu.async_copy(
            t_hbm.at[pl.ds(r, R_SUB)], tbufs[slot], stab[slot])
        for b in range(B):
            in_d[g][b] = pltpu.async_copy(
                x_hbm.at[b, pl.ds(r, R_SUB)], xbufs[slot][b], sin[slot][b])

    issue_group(0)
    issue_group(1)

    for g in range(NG):
        slot = g % NRING
        t_d[g].wait()
        for b in range(B):
            in_d[g][b].wait()
        tb = tbufs[slot]
        xa, xbb, xc, xd = xbufs[slot]

        @plsc.parallel_loop(0, R_SUB * VPR, step=1, unroll=4)
        def add_loop(i):
            row = i >> 6
            col = (i & (VPR - 1)) * L
            v = tb[row, pl.ds(col, L)]
            plsc.addupdate(xa.at[row, pl.ds(col, L)], v)
            plsc.addupdate(xbb.at[row, pl.ds(col, L)], v)
            plsc.addupdate(xc.at[row, pl.ds(col, L)], v)
            plsc.addupdate(xd.at[row, pl.ds(col, L)], v)

        gn = g + 2
        if gn < NG:
            if gn - NRING >= 0:
                for b in range(B):
                    out_d[gn - NRING][b].wait()
            issue_group(gn)

        r = row0 + g * R_SUB
        for b in range(B):
            out_d[g][b] = pltpu.async_copy(
                xbufs[slot][b], o_hbm.at[b, pl.ds(r, R_SUB)], sout[slot][b])

    for g in range(max(0, NG - NRING), NG):
        for b in range(B):
            if out_d[g][b] is not None:
                out_d[g][b].wait()


_sc_call = functools.partial(
    pl.kernel,
    mesh=plsc.VectorSubcoreMesh(core_axis_name="c", subcore_axis_name="s"),
    out_type=jax.ShapeDtypeStruct((B, S, D), jnp.float32),
    scratch_types=(
        [pltpu.VMEM((R_SUB, D), jnp.float32)] * (NRING * B + NRING)
        + [pltpu.SemaphoreType.DMA] * (2 * NRING * B + NRING)
    ),
)


def kernel(x, pos_table):
    batch, seq_len, d_model = x.shape
    return _sc_call(_sc_body)(x, pos_table[:seq_len])
